# baseline (speedup 1.0000x reference)
import functools

import jax
import jax.numpy as jnp
from jax import lax
from jax.experimental import pallas as pl
from jax.experimental.pallas import tpu as pltpu
from jax.experimental.pallas import tpu_sc as plsc

N = 10000
E = 160000
DF = 256
DL = 512

NC = 2
NS = 16
LANES = 16
QW = 64
NQ = 4

NPAD = 10240
EPAD = 163840
CHUNK = 128
CHUNK_C = 80
NBUF = 4
CA = EPAD // (NC * NS) // CHUNK
CC = EPAD // NS // CHUNK_C
RPT = NPAD // NS

_mesh = plsc.VectorSubcoreMesh(core_axis_name="c", subcore_axis_name="s")
_sc_params = pltpu.CompilerParams(
    needs_layout_passes=False, use_tc_tiling_on_sc=False
)


@functools.partial(
    pl.kernel,
    out_type=jax.ShapeDtypeStruct((NC * NS, NPAD), jnp.float32),
    mesh=_mesh,
    compiler_params=_sc_params,
    scratch_types=[
        pltpu.VMEM((CA, CHUNK), jnp.int32),
        pltpu.VMEM((NPAD,), jnp.float32),
    ],
)
def _deg_kernel(dst_hbm, deg_out, idx_v, hist_v):
    cid = lax.axis_index("c")
    sid = lax.axis_index("s")
    wid = sid * NC + cid

    @pl.loop(0, NPAD // LANES)
    def _zero(i):
        hist_v[pl.ds(i * LANES, LANES)] = jnp.zeros((LANES,), jnp.float32)

    pltpu.sync_copy(dst_hbm.at[wid], idx_v)
    ones = jnp.full((LANES,), 1.0, jnp.float32)

    @pl.loop(0, CA)
    def _chunk(j):
        @pl.loop(0, CHUNK // LANES)
        def _vec(k):
            idx = idx_v[j, pl.ds(k * LANES, LANES)]
            plsc.addupdate_scatter(hist_v, [idx], ones)

    pltpu.sync_copy(hist_v, deg_out.at[wid])


def _prep_body(parts_ref, x_ref, dis_ref, x4_ref):
    deg = jnp.sum(parts_ref[...], axis=0) + 1.0
    dis = jax.lax.rsqrt(deg)[:, None]
    dis_ref[...] = dis
    for q in range(NQ):
        x4_ref[q] = x_ref[:, q * QW:(q + 1) * QW] * dis


def _prep(parts, x_pad):
    blk = 1024
    grid = NPAD // blk
    return pl.pallas_call(
        _prep_body,
        grid=(grid,),
        in_specs=[
            pl.BlockSpec((NC * NS, blk), lambda i: (0, i)),
            pl.BlockSpec((blk, DF), lambda i: (i, 0)),
        ],
        out_specs=[
            pl.BlockSpec((blk, 1), lambda i: (i, 0)),
            pl.BlockSpec((NQ, blk, QW), lambda i: (0, i, 0)),
        ],
        out_shape=[
            jax.ShapeDtypeStruct((NPAD, 1), jnp.float32),
            jax.ShapeDtypeStruct((NQ, NPAD, QW), jnp.float32),
        ],
    )(parts, x_pad)


@functools.partial(
    pl.kernel,
    out_type=jax.ShapeDtypeStruct((NQ, NPAD, QW), jnp.float32),
    mesh=_mesh,
    compiler_params=_sc_params,
    scratch_types=[
        pltpu.VMEM((CC, CHUNK_C), jnp.int32),
        pltpu.VMEM((CC, CHUNK_C), jnp.int32),
        [pltpu.VMEM((CHUNK_C, QW), jnp.float32)] * NBUF,
        [pltpu.SemaphoreType.DMA] * NBUF,
        [pltpu.SemaphoreType.DMA] * NBUF,
        pltpu.VMEM_SHARED((NPAD, QW), jnp.float32),
        pltpu.VMEM_SHARED((NPAD, QW), jnp.float32),
    ],
)
def _agg_kernel(x4_hbm, src_hbm, dst_hbm, s_out, src_v, dst_v, gbufs, gsems,
                ssems, x_sh, s_sh):
    cid = lax.axis_index("c")
    sid = lax.axis_index("s")

    pltpu.sync_copy(src_hbm.at[sid], src_v)
    pltpu.sync_copy(dst_hbm.at[sid], dst_v)

    def _gather(c, b):
        return pltpu.make_async_copy(x_sh.at[src_v.at[c]], gbufs[b], gsems[b])

    def _scatter(c, b):
        return pltpu.make_async_copy(gbufs[b], s_sh.at[dst_v.at[c]], ssems[b])

    for p in range(2):
        q = 2 * cid + p
        pltpu.sync_copy(
            x4_hbm.at[pl.ds(q * NPAD + sid * RPT, RPT)],
            x_sh.at[pl.ds(sid * RPT, RPT)],
        )
        pltpu.sync_copy(
            x4_hbm.at[pl.ds(q * NPAD + sid * RPT, RPT)],
            s_sh.at[pl.ds(sid * RPT, RPT)],
        )
        plsc.subcore_barrier()

        pltpu.async_copy(x_sh.at[src_v.at[0]], gbufs[0], gsems[0])
        pltpu.async_copy(x_sh.at[src_v.at[1]], gbufs[1], gsems[1])

        @pl.loop(0, CC, step=NBUF)
        def _edges(j):
            for b in range(NBUF):
                c = j + b
                nb = (b + 2) % NBUF

                @pl.when(c + 2 < CC)
                def _refill():
                    @pl.when(c >= 2)
                    def _retire():
                        _scatter(c - 2, nb).wait()

                    pltpu.async_copy(
                        x_sh.at[src_v.at[c + 2]], gbufs[nb], gsems[nb]
                    )

                _gather(c, b).wait()
                pltpu.async_copy(
                    gbufs[b], s_sh.at[dst_v.at[c]], ssems[b], add=True
                )

        for t in range(CC - NBUF, CC):
            _scatter(t, t % NBUF).wait()

        plsc.subcore_barrier()
        pltpu.sync_copy(
            s_sh.at[pl.ds(sid * RPT, RPT)],
            s_out.at[q, pl.ds(sid * RPT, RPT)],
        )
        plsc.subcore_barrier()


def _mm_body(s_ref, dis_ref, w_ref, b_ref, out_ref):
    dis = dis_ref[...]
    acc = jnp.broadcast_to(b_ref[...], out_ref.shape).astype(jnp.float32)
    for q in range(NQ):
        acc = acc + jnp.dot(
            s_ref[q] * dis, w_ref[q], preferred_element_type=jnp.float32
        )
    out_ref[...] = acc


def _matmul(s, dis, w4, b):
    blk = 512
    grid = NPAD // blk
    return pl.pallas_call(
        _mm_body,
        grid=(grid,),
        in_specs=[
            pl.BlockSpec((NQ, blk, QW), lambda i: (0, i, 0)),
            pl.BlockSpec((blk, 1), lambda i: (i, 0)),
            pl.BlockSpec((NQ, QW, DL), lambda i: (0, 0, 0)),
            pl.BlockSpec((1, DL), lambda i: (0, 0)),
        ],
        out_specs=pl.BlockSpec((blk, DL), lambda i: (i, 0)),
        out_shape=jax.ShapeDtypeStruct((NPAD, DL), jnp.float32),
    )(s, dis, w4, b)


def kernel(x, reg_edge_index, W_mu, b_mu):
    src = reg_edge_index[0].astype(jnp.int32)
    dst = reg_edge_index[1].astype(jnp.int32)
    pad = EPAD - E
    srcp = jnp.concatenate([src, jnp.zeros((pad,), jnp.int32)])
    dstp = jnp.concatenate([dst, jnp.full((pad,), N, jnp.int32)])

    dst_a = dstp.reshape(NC * NS, CA, CHUNK)
    src_c = srcp.reshape(NS, CC, CHUNK_C)
    dst_c = dstp.reshape(NS, CC, CHUNK_C)

    x_pad = jnp.pad(x, ((0, NPAD - N), (0, 0)))

    parts = _deg_kernel(dst_a)
    dis, x4 = _prep(parts, x_pad)
    s = _agg_kernel(x4.reshape(NQ * NPAD, QW), src_c, dst_c)
    out = _matmul(s, dis, W_mu.reshape(NQ, QW, DL), b_mu[None, :])
    return out[:N]


# speedup vs baseline: 1.3379x; 1.3379x over previous
import functools

import jax
import jax.numpy as jnp
from jax import lax
from jax.experimental import pallas as pl
from jax.experimental.pallas import tpu as pltpu
from jax.experimental.pallas import tpu_sc as plsc

N = 10000
E = 160000
DF = 256
DL = 512

NC = 2
NS = 16
LANES = 16
QW = 128
NQ = 2

NPAD = 10240
EPAD = 163840
CHUNK = 128
CHUNK_C = 80
NBUF = 4
CA = EPAD // (NC * NS) // CHUNK
CC = EPAD // NS // CHUNK_C
RPT = NPAD // NS

_mesh = plsc.VectorSubcoreMesh(core_axis_name="c", subcore_axis_name="s")
_sc_params = pltpu.CompilerParams(
    needs_layout_passes=False, use_tc_tiling_on_sc=False
)


@functools.partial(
    pl.kernel,
    out_type=jax.ShapeDtypeStruct((NC * NS, NPAD), jnp.float32),
    mesh=_mesh,
    compiler_params=_sc_params,
    scratch_types=[
        pltpu.VMEM((CA, CHUNK), jnp.int32),
        pltpu.VMEM((NPAD,), jnp.float32),
    ],
)
def _deg_kernel(dst_hbm, deg_out, idx_v, hist_v):
    cid = lax.axis_index("c")
    sid = lax.axis_index("s")
    wid = sid * NC + cid

    @pl.loop(0, NPAD // LANES)
    def _zero(i):
        hist_v[pl.ds(i * LANES, LANES)] = jnp.zeros((LANES,), jnp.float32)

    pltpu.sync_copy(dst_hbm.at[wid], idx_v)
    ones = jnp.full((LANES,), 1.0, jnp.float32)

    @pl.loop(0, CA)
    def _chunk(j):
        @pl.loop(0, CHUNK // LANES)
        def _vec(k):
            idx = idx_v[j, pl.ds(k * LANES, LANES)]
            plsc.addupdate_scatter(hist_v, [idx], ones)

    pltpu.sync_copy(hist_v, deg_out.at[wid])


def _prep_body(parts_ref, x_ref, dis_ref, x4_ref):
    deg = jnp.sum(parts_ref[...], axis=0) + 1.0
    dis = jax.lax.rsqrt(deg)[:, None]
    dis_ref[...] = dis
    for q in range(NQ):
        x4_ref[q] = (x_ref[:, q * QW:(q + 1) * QW] * dis).astype(jnp.bfloat16)


def _prep(parts, x_pad):
    blk = 1024
    grid = NPAD // blk
    return pl.pallas_call(
        _prep_body,
        grid=(grid,),
        in_specs=[
            pl.BlockSpec((NC * NS, blk), lambda i: (0, i)),
            pl.BlockSpec((blk, DF), lambda i: (i, 0)),
        ],
        out_specs=[
            pl.BlockSpec((blk, 1), lambda i: (i, 0)),
            pl.BlockSpec((NQ, blk, QW), lambda i: (0, i, 0)),
        ],
        out_shape=[
            jax.ShapeDtypeStruct((NPAD, 1), jnp.float32),
            jax.ShapeDtypeStruct((NQ, NPAD, QW), jnp.bfloat16),
        ],
    )(parts, x_pad)


@functools.partial(
    pl.kernel,
    out_type=jax.ShapeDtypeStruct((NQ, NPAD, QW), jnp.bfloat16),
    mesh=_mesh,
    compiler_params=_sc_params,
    scratch_types=[
        pltpu.VMEM((CC, CHUNK_C), jnp.int32),
        pltpu.VMEM((CC, CHUNK_C), jnp.int32),
        [pltpu.VMEM((CHUNK_C, QW), jnp.bfloat16)] * NBUF,
        [pltpu.SemaphoreType.DMA] * NBUF,
        [pltpu.SemaphoreType.DMA] * NBUF,
        pltpu.VMEM_SHARED((NPAD, QW), jnp.bfloat16),
        pltpu.VMEM_SHARED((NPAD, QW), jnp.bfloat16),
    ],
)
def _agg_kernel(x4_hbm, src_hbm, dst_hbm, s_out, src_v, dst_v, gbufs, gsems,
                ssems, x_sh, s_sh):
    cid = lax.axis_index("c")
    sid = lax.axis_index("s")

    pltpu.sync_copy(src_hbm.at[sid], src_v)
    pltpu.sync_copy(dst_hbm.at[sid], dst_v)

    def _gather(c, b):
        return pltpu.make_async_copy(x_sh.at[src_v.at[c]], gbufs[b], gsems[b])

    def _scatter(c, b):
        return pltpu.make_async_copy(gbufs[b], s_sh.at[dst_v.at[c]], ssems[b])

    for p in range(1):
        q = cid
        pltpu.sync_copy(
            x4_hbm.at[pl.ds(q * NPAD + sid * RPT, RPT)],
            x_sh.at[pl.ds(sid * RPT, RPT)],
        )
        pltpu.sync_copy(
            x4_hbm.at[pl.ds(q * NPAD + sid * RPT, RPT)],
            s_sh.at[pl.ds(sid * RPT, RPT)],
        )
        plsc.subcore_barrier()

        pltpu.async_copy(x_sh.at[src_v.at[0]], gbufs[0], gsems[0])
        pltpu.async_copy(x_sh.at[src_v.at[1]], gbufs[1], gsems[1])

        @pl.loop(0, CC, step=NBUF)
        def _edges(j):
            for b in range(NBUF):
                c = j + b
                nb = (b + 2) % NBUF

                @pl.when(c + 2 < CC)
                def _refill():
                    @pl.when(c >= 2)
                    def _retire():
                        _scatter(c - 2, nb).wait()

                    pltpu.async_copy(
                        x_sh.at[src_v.at[c + 2]], gbufs[nb], gsems[nb]
                    )

                _gather(c, b).wait()
                pltpu.async_copy(
                    gbufs[b], s_sh.at[dst_v.at[c]], ssems[b], add=True
                )

        for t in range(CC - NBUF, CC):
            _scatter(t, t % NBUF).wait()

        plsc.subcore_barrier()
        pltpu.sync_copy(
            s_sh.at[pl.ds(sid * RPT, RPT)],
            s_out.at[q, pl.ds(sid * RPT, RPT)],
        )
        plsc.subcore_barrier()


def _mm_body(s_ref, dis_ref, w_ref, b_ref, out_ref):
    dis = dis_ref[...]
    acc = jnp.broadcast_to(b_ref[...], out_ref.shape).astype(jnp.float32)
    for q in range(NQ):
        acc = acc + jnp.dot(
            s_ref[q].astype(jnp.float32) * dis, w_ref[q],
            preferred_element_type=jnp.float32,
        )
    out_ref[...] = acc


def _matmul(s, dis, w4, b):
    blk = 512
    grid = NPAD // blk
    return pl.pallas_call(
        _mm_body,
        grid=(grid,),
        in_specs=[
            pl.BlockSpec((NQ, blk, QW), lambda i: (0, i, 0)),
            pl.BlockSpec((blk, 1), lambda i: (i, 0)),
            pl.BlockSpec((NQ, QW, DL), lambda i: (0, 0, 0)),
            pl.BlockSpec((1, DL), lambda i: (0, 0)),
        ],
        out_specs=pl.BlockSpec((blk, DL), lambda i: (i, 0)),
        out_shape=jax.ShapeDtypeStruct((NPAD, DL), jnp.float32),
    )(s, dis, w4, b)


def kernel(x, reg_edge_index, W_mu, b_mu):
    src = reg_edge_index[0].astype(jnp.int32)
    dst = reg_edge_index[1].astype(jnp.int32)
    pad = EPAD - E
    srcp = jnp.concatenate([src, jnp.zeros((pad,), jnp.int32)])
    dstp = jnp.concatenate([dst, jnp.full((pad,), N, jnp.int32)])

    dst_a = dstp.reshape(NC * NS, CA, CHUNK)
    src_c = srcp.reshape(NS, CC, CHUNK_C)
    dst_c = dstp.reshape(NS, CC, CHUNK_C)

    x_pad = jnp.pad(x, ((0, NPAD - N), (0, 0)))

    parts = _deg_kernel(dst_a)
    dis, x4 = _prep(parts, x_pad)
    s = _agg_kernel(x4.reshape(NQ * NPAD, QW), src_c, dst_c)
    out = _matmul(s, dis, W_mu.reshape(NQ, QW, DL), b_mu[None, :])
    return out[:N]


# speedup vs baseline: 1.4920x; 1.1152x over previous
import functools

import jax
import jax.numpy as jnp
from jax import lax
from jax.experimental import pallas as pl
from jax.experimental.pallas import tpu as pltpu
from jax.experimental.pallas import tpu_sc as plsc

N = 10000
E = 160000
DF = 256
DL = 512

NC = 2
NS = 16
LANES = 16
QW = 128
NQ = 2

NPAD = 10240
EPAD = 163840
CHUNK = 128
CHUNK_C = 80
NBUF = 4
CA = EPAD // (NC * NS) // CHUNK
CC = EPAD // NS // CHUNK_C
RPT = NPAD // NS

_mesh = plsc.VectorSubcoreMesh(core_axis_name="c", subcore_axis_name="s")
_sc_params = pltpu.CompilerParams(
    needs_layout_passes=False, use_tc_tiling_on_sc=False
)


@functools.partial(
    pl.kernel,
    out_type=jax.ShapeDtypeStruct((NC * NS, NPAD), jnp.float32),
    mesh=_mesh,
    compiler_params=_sc_params,
    scratch_types=[
        pltpu.VMEM((CA, CHUNK), jnp.int32),
        pltpu.VMEM((NPAD,), jnp.float32),
    ],
)
def _deg_kernel(dst_hbm, deg_out, idx_v, hist_v):
    cid = lax.axis_index("c")
    sid = lax.axis_index("s")
    wid = sid * NC + cid

    @pl.loop(0, NPAD // LANES)
    def _zero(i):
        hist_v[pl.ds(i * LANES, LANES)] = jnp.zeros((LANES,), jnp.float32)

    pltpu.sync_copy(dst_hbm.at[wid], idx_v)
    ones = jnp.full((LANES,), 1.0, jnp.float32)

    @pl.loop(0, CA)
    def _chunk(j):
        @pl.loop(0, CHUNK // LANES)
        def _vec(k):
            idx = idx_v[j, pl.ds(k * LANES, LANES)]
            plsc.addupdate_scatter(hist_v, [idx], ones)

    pltpu.sync_copy(hist_v, deg_out.at[wid])


def _prep_body(parts_ref, x_ref, dis_ref, x4_ref):
    deg = jnp.sum(parts_ref[...], axis=0) + 1.0
    dis = jax.lax.rsqrt(deg)[:, None]
    dis_ref[...] = dis
    for q in range(NQ):
        x4_ref[q] = (x_ref[:, q * QW:(q + 1) * QW] * dis).astype(jnp.bfloat16)


def _prep(parts, x):
    blk = 1024
    grid = NPAD // blk
    return pl.pallas_call(
        _prep_body,
        grid=(grid,),
        in_specs=[
            pl.BlockSpec((NC * NS, blk), lambda i: (0, i)),
            pl.BlockSpec((blk, DF), lambda i: (i, 0)),
        ],
        out_specs=[
            pl.BlockSpec((blk, 1), lambda i: (i, 0)),
            pl.BlockSpec((NQ, blk, QW), lambda i: (0, i, 0)),
        ],
        out_shape=[
            jax.ShapeDtypeStruct((NPAD, 1), jnp.float32),
            jax.ShapeDtypeStruct((NQ, NPAD, QW), jnp.bfloat16),
        ],
    )(parts, x)


@functools.partial(
    pl.kernel,
    out_type=jax.ShapeDtypeStruct((NQ, NPAD, QW), jnp.bfloat16),
    mesh=_mesh,
    compiler_params=_sc_params,
    scratch_types=[
        pltpu.VMEM((CC, CHUNK_C), jnp.int32),
        pltpu.VMEM((CC, CHUNK_C), jnp.int32),
        [pltpu.VMEM((CHUNK_C, QW), jnp.bfloat16)] * NBUF,
        [pltpu.SemaphoreType.DMA] * NBUF,
        [pltpu.SemaphoreType.DMA] * NBUF,
        pltpu.VMEM_SHARED((NPAD, QW), jnp.bfloat16),
        pltpu.VMEM_SHARED((NPAD, QW), jnp.bfloat16),
    ],
)
def _agg_kernel(x4_hbm, src_hbm, dst_hbm, s_out, src_v, dst_v, gbufs, gsems,
                ssems, x_sh, s_sh):
    cid = lax.axis_index("c")
    sid = lax.axis_index("s")

    pltpu.sync_copy(src_hbm.at[sid], src_v)
    pltpu.sync_copy(dst_hbm.at[sid], dst_v)

    def _gather(c, b):
        return pltpu.make_async_copy(x_sh.at[src_v.at[c]], gbufs[b], gsems[b])

    def _scatter(c, b):
        return pltpu.make_async_copy(gbufs[b], s_sh.at[dst_v.at[c]], ssems[b])

    for p in range(1):
        q = cid
        pltpu.sync_copy(
            x4_hbm.at[q, pl.ds(sid * RPT, RPT)],
            x_sh.at[pl.ds(sid * RPT, RPT)],
        )
        pltpu.sync_copy(
            x4_hbm.at[q, pl.ds(sid * RPT, RPT)],
            s_sh.at[pl.ds(sid * RPT, RPT)],
        )
        plsc.subcore_barrier()

        pltpu.async_copy(x_sh.at[src_v.at[0]], gbufs[0], gsems[0])
        pltpu.async_copy(x_sh.at[src_v.at[1]], gbufs[1], gsems[1])

        @pl.loop(0, CC, step=NBUF)
        def _edges(j):
            for b in range(NBUF):
                c = j + b
                nb = (b + 2) % NBUF

                @pl.when(c + 2 < CC)
                def _refill():
                    @pl.when(c >= 2)
                    def _retire():
                        _scatter(c - 2, nb).wait()

                    pltpu.async_copy(
                        x_sh.at[src_v.at[c + 2]], gbufs[nb], gsems[nb]
                    )

                _gather(c, b).wait()
                pltpu.async_copy(
                    gbufs[b], s_sh.at[dst_v.at[c]], ssems[b], add=True
                )

        for t in range(CC - NBUF, CC):
            _scatter(t, t % NBUF).wait()

        plsc.subcore_barrier()
        pltpu.sync_copy(
            s_sh.at[pl.ds(sid * RPT, RPT)],
            s_out.at[q, pl.ds(sid * RPT, RPT)],
        )
        plsc.subcore_barrier()


def _mm_body(s_ref, dis_ref, w_ref, b_ref, out_ref):
    w = w_ref[...]
    acc = jnp.dot(
        s_ref[0].astype(jnp.float32), w[:QW], preferred_element_type=jnp.float32
    )
    acc = acc + jnp.dot(
        s_ref[1].astype(jnp.float32), w[QW:], preferred_element_type=jnp.float32
    )
    out_ref[...] = acc * dis_ref[...] + b_ref[...]


def _matmul(s, dis, w, b):
    blk = 1000
    grid = N // blk
    return pl.pallas_call(
        _mm_body,
        grid=(grid,),
        in_specs=[
            pl.BlockSpec((NQ, blk, QW), lambda i: (0, i, 0)),
            pl.BlockSpec((blk, 1), lambda i: (i, 0)),
            pl.BlockSpec((DF, DL), lambda i: (0, 0)),
            pl.BlockSpec((1, DL), lambda i: (0, 0)),
        ],
        out_specs=pl.BlockSpec((blk, DL), lambda i: (i, 0)),
        out_shape=jax.ShapeDtypeStruct((N, DL), jnp.float32),
    )(s, dis, w, b)


def kernel(x, reg_edge_index, W_mu, b_mu):
    src = reg_edge_index[0].astype(jnp.int32)
    dst = reg_edge_index[1].astype(jnp.int32)
    pad = EPAD - E
    srcp = jnp.concatenate([src, jnp.zeros((pad,), jnp.int32)])
    dstp = jnp.concatenate([dst, jnp.full((pad,), N, jnp.int32)])

    dst_a = dstp.reshape(NC * NS, CA, CHUNK)
    src_c = srcp.reshape(NS, CC, CHUNK_C)
    dst_c = dstp.reshape(NS, CC, CHUNK_C)

    parts = _deg_kernel(dst_a)
    dis, x4 = _prep(parts, x)
    s = _agg_kernel(x4, src_c, dst_c)
    return _matmul(s, dis, W_mu, b_mu.reshape(1, DL))
